# Initial kernel scaffold; baseline (speedup 1.0000x reference)
import jax
import jax.numpy as jnp
from jax.experimental import pallas as pl


def kernel(x, embedding_weight):
    raise NotImplementedError("write your pallas kernel here")


# speedup vs baseline: 1.7085x; 1.7085x over previous
import functools

import jax
import jax.numpy as jnp
from jax import lax
from jax.experimental import pallas as pl
from jax.experimental.pallas import tpu as pltpu
from jax.experimental.pallas import tpu_sc as plsc

EMB_DIM = 128
NC, NS = 2, 16
NW = NC * NS
K = 5
CHUNK = K * 128


@functools.cache
def _gather_call(batch):
    per_w = batch // NW
    n_chunks = per_w // CHUNK
    mesh = plsc.VectorSubcoreMesh(
        core_axis_name="c", subcore_axis_name="s",
        num_cores=NC, num_subcores=NS,
    )

    @functools.partial(
        pl.kernel,
        out_type=jax.ShapeDtypeStruct((batch, EMB_DIM), jnp.float32),
        mesh=mesh,
        scratch_types=[
            pltpu.VMEM((CHUNK,), jnp.int32),
            pltpu.VMEM((CHUNK, EMB_DIM), jnp.float32),
            pltpu.SemaphoreType.DMA,
        ],
    )
    def k(table_hbm, idx_hbm, out_hbm, idx_v, rows_v, sem):
        wid = lax.axis_index("s") * NC + lax.axis_index("c")
        wbase = wid * per_w

        def body(i, carry):
            base = wbase + i * CHUNK
            pltpu.sync_copy(idx_hbm.at[pl.ds(base, CHUNK)], idx_v)
            copies = [
                pltpu.async_copy(
                    table_hbm.at[idx_v.at[pl.ds(j * 128, 128)]],
                    rows_v.at[pl.ds(j * 128, 128)],
                    sem,
                )
                for j in range(K)
            ]
            for c in copies:
                c.wait()
            pltpu.sync_copy(rows_v, out_hbm.at[pl.ds(base, CHUNK)])
            return carry

        lax.fori_loop(0, n_chunks, body, 0)

    return k


def kernel(x, embedding_weight):
    batch = x.size
    idx = x.reshape(batch)
    out = _gather_call(batch)(embedding_weight, idx)
    return out.reshape(x.shape + (EMB_DIM,))


# speedup vs baseline: 1.8546x; 1.0855x over previous
import functools

import jax
import jax.numpy as jnp
from jax import lax
from jax.experimental import pallas as pl
from jax.experimental.pallas import tpu as pltpu
from jax.experimental.pallas import tpu_sc as plsc

EMB_DIM = 128
NC, NS = 2, 16
NW = NC * NS
C = 128
NB = 5
D = 3


@functools.cache
def _gather_call(batch):
    per_w = batch // NW
    n_chunks = per_w // C
    n_groups = n_chunks // NB
    assert per_w % C == 0 and n_chunks % NB == 0 and n_groups >= 3

    mesh = plsc.VectorSubcoreMesh(
        core_axis_name="c", subcore_axis_name="s",
        num_cores=NC, num_subcores=NS,
    )

    @functools.partial(
        pl.kernel,
        out_type=jax.ShapeDtypeStruct((batch, EMB_DIM), jnp.float32),
        mesh=mesh,
        scratch_types=(
            [pltpu.VMEM((per_w,), jnp.int32)]
            + [pltpu.VMEM((C, EMB_DIM), jnp.float32) for _ in range(NB)]
            + [pltpu.SemaphoreType.DMA] * (2 * NB)
        ),
    )
    def k(table_hbm, idx_hbm, out_hbm, idx_v, *rest):
        rows = rest[:NB]
        gsem = rest[NB:2 * NB]
        wsem = rest[2 * NB:]
        wid = lax.axis_index("s") * NC + lax.axis_index("c")
        wbase = wid * per_w

        pltpu.sync_copy(idx_hbm.at[pl.ds(wbase, per_w)], idx_v)

        def fire_gather(c, b):
            pltpu.async_copy(
                table_hbm.at[idx_v.at[pl.ds(c * C, C)]], rows[b], gsem[b])

        def wait_gather(b):
            pltpu.make_async_copy(
                table_hbm.at[idx_v.at[pl.ds(0, C)]], rows[b], gsem[b]).wait()

        def fire_write(c, b):
            pltpu.async_copy(
                rows[b], out_hbm.at[pl.ds(wbase + c * C, C)], wsem[b])

        def wait_write(b):
            pltpu.make_async_copy(
                rows[b], out_hbm.at[pl.ds(wbase, C)], wsem[b]).wait()

        def step(j, b, issue, reuse_wait):
            bi = (b + D) % NB
            if issue:
                if reuse_wait:
                    wait_write(bi)
                fire_gather(j + D, bi)
            wait_gather(b)
            fire_write(j, b)

        for b in range(D):
            fire_gather(b, b)

        for b in range(NB):
            step(b, b, issue=True, reuse_wait=(b >= NB - D))

        def group_body(g, carry):
            for b in range(NB):
                step(g * NB + b, b, issue=True, reuse_wait=True)
            return carry

        lax.fori_loop(1, n_groups - 1, group_body, 0)

        for b in range(NB):
            j = n_chunks - NB + b
            step(j, b, issue=(b < NB - D), reuse_wait=True)

        for b in range(NB):
            wait_write(b)

    return k


def kernel(x, embedding_weight):
    batch = x.size
    idx = x.reshape(batch)
    out = _gather_call(batch)(embedding_weight, idx)
    return out.reshape(x.shape + (EMB_DIM,))


# speedup vs baseline: 1.8590x; 1.0023x over previous
import functools

import jax
import jax.numpy as jnp
from jax import lax
from jax.experimental import pallas as pl
from jax.experimental.pallas import tpu as pltpu
from jax.experimental.pallas import tpu_sc as plsc

EMB_DIM = 128
NC, NS = 2, 16
NW = NC * NS
C = 128
NB = 5
D = 4


@functools.cache
def _gather_call(batch):
    per_w = batch // NW
    n_chunks = per_w // C
    n_groups = n_chunks // NB
    assert per_w % C == 0 and n_chunks % NB == 0 and n_groups >= 3

    mesh = plsc.VectorSubcoreMesh(
        core_axis_name="c", subcore_axis_name="s",
        num_cores=NC, num_subcores=NS,
    )

    @functools.partial(
        pl.kernel,
        out_type=jax.ShapeDtypeStruct((batch, EMB_DIM), jnp.float32),
        mesh=mesh,
        scratch_types=(
            [pltpu.VMEM((per_w,), jnp.int32)]
            + [pltpu.VMEM((C, EMB_DIM), jnp.float32) for _ in range(NB)]
            + [pltpu.SemaphoreType.DMA] * (2 * NB)
        ),
    )
    def k(table_hbm, idx_hbm, out_hbm, idx_v, *rest):
        rows = rest[:NB]
        gsem = rest[NB:2 * NB]
        wsem = rest[2 * NB:]
        wid = lax.axis_index("s") * NC + lax.axis_index("c")
        wbase = wid * per_w

        pltpu.sync_copy(idx_hbm.at[pl.ds(wbase, per_w)], idx_v)

        def fire_gather(c, b):
            pltpu.async_copy(
                table_hbm.at[idx_v.at[pl.ds(c * C, C)]], rows[b], gsem[b])

        def wait_gather(b):
            pltpu.make_async_copy(
                table_hbm.at[idx_v.at[pl.ds(0, C)]], rows[b], gsem[b]).wait()

        def fire_write(c, b):
            pltpu.async_copy(
                rows[b], out_hbm.at[pl.ds(wbase + c * C, C)], wsem[b])

        def wait_write(b):
            pltpu.make_async_copy(
                rows[b], out_hbm.at[pl.ds(wbase, C)], wsem[b]).wait()

        def step(j, b, issue, reuse_wait):
            bi = (b + D) % NB
            if issue:
                if reuse_wait:
                    wait_write(bi)
                fire_gather(j + D, bi)
            wait_gather(b)
            fire_write(j, b)

        for b in range(D):
            fire_gather(b, b)

        for b in range(NB):
            step(b, b, issue=True, reuse_wait=(b >= NB - D))

        def group_body(g, carry):
            for b in range(NB):
                step(g * NB + b, b, issue=True, reuse_wait=True)
            return carry

        lax.fori_loop(1, n_groups - 1, group_body, 0)

        for b in range(NB):
            j = n_chunks - NB + b
            step(j, b, issue=(b < NB - D), reuse_wait=True)

        for b in range(NB):
            wait_write(b)

    return k


def kernel(x, embedding_weight):
    batch = x.size
    idx = x.reshape(batch)
    out = _gather_call(batch)(embedding_weight, idx)
    return out.reshape(x.shape + (EMB_DIM,))


# speedup vs baseline: 1.8590x; 1.0000x over previous
import functools

import jax
import jax.numpy as jnp
from jax import lax
from jax.experimental import pallas as pl
from jax.experimental.pallas import tpu as pltpu
from jax.experimental.pallas import tpu_sc as plsc

EMB_DIM = 128
NC, NS = 2, 16
NW = NC * NS
G = 128
K = 2
C = K * G
NB = 3
D = 2


@functools.cache
def _gather_call(batch):
    per_w = batch // NW
    n_chunks = per_w // C
    assert per_w % C == 0 and n_chunks > 2 * NB

    mesh = plsc.VectorSubcoreMesh(
        core_axis_name="c", subcore_axis_name="s",
        num_cores=NC, num_subcores=NS,
    )

    @functools.partial(
        pl.kernel,
        out_type=jax.ShapeDtypeStruct((batch, EMB_DIM), jnp.float32),
        mesh=mesh,
        scratch_types=(
            [pltpu.VMEM((per_w,), jnp.int32)]
            + [pltpu.VMEM((C, EMB_DIM), jnp.float32) for _ in range(NB)]
            + [pltpu.SemaphoreType.DMA] * (2 * NB)
        ),
    )
    def k(table_hbm, idx_hbm, out_hbm, idx_v, *rest):
        rows = rest[:NB]
        gsem = rest[NB:2 * NB]
        wsem = rest[2 * NB:]
        wid = lax.axis_index("s") * NC + lax.axis_index("c")
        wbase = wid * per_w

        pltpu.sync_copy(idx_hbm.at[pl.ds(wbase, per_w)], idx_v)

        def fire_gathers(c, b):
            for j in range(K):
                pltpu.async_copy(
                    table_hbm.at[idx_v.at[pl.ds(c * C + j * G, G)]],
                    rows[b].at[pl.ds(j * G, G)], gsem[b])

        def wait_gathers(b):
            for j in range(K):
                pltpu.make_async_copy(
                    table_hbm.at[idx_v.at[pl.ds(0, G)]],
                    rows[b].at[pl.ds(j * G, G)], gsem[b]).wait()

        def fire_write(c, b):
            pltpu.async_copy(
                rows[b], out_hbm.at[pl.ds(wbase + c * C, C)], wsem[b])

        def wait_write(b):
            pltpu.make_async_copy(
                rows[b], out_hbm.at[pl.ds(wbase, C)], wsem[b]).wait()

        def step(j, b, issue, reuse_wait):
            bi = (b + D) % NB
            if issue:
                if reuse_wait:
                    wait_write(bi)
                fire_gathers(j + D, bi)
            wait_gathers(b)
            fire_write(j, b)

        for b in range(D):
            fire_gathers(b, b)

        for b in range(NB):
            step(b, b, issue=True, reuse_wait=(b >= NB - D))

        n_mid_groups = (n_chunks - D) // NB - 1
        mid_end = NB + n_mid_groups * NB

        def group_body(g, carry):
            for b in range(NB):
                step(g * NB + b, b, issue=True, reuse_wait=True)
            return carry

        lax.fori_loop(1, 1 + n_mid_groups, group_body, 0)

        for j in range(mid_end, n_chunks):
            step(j, j % NB, issue=(j + D < n_chunks), reuse_wait=True)

        for b in range(NB):
            wait_write(b)

    return k


def kernel(x, embedding_weight):
    batch = x.size
    idx = x.reshape(batch)
    out = _gather_call(batch)(embedding_weight, idx)
    return out.reshape(x.shape + (EMB_DIM,))
